# Initial kernel scaffold; baseline (speedup 1.0000x reference)
import jax
import jax.numpy as jnp
from jax.experimental import pallas as pl


def kernel(features, graph, fc0_W, fc0_b, conv_W, fc1_W, fc1_b):
    raise NotImplementedError("write your pallas kernel here")


# speedup vs baseline: 2.9120x; 2.9120x over previous
import functools
import math

import jax
import jax.numpy as jnp
from jax import lax
from jax.experimental import pallas as pl
from jax.experimental.pallas import tpu as pltpu
from jax.experimental.pallas import tpu_sc as plsc

N = 10000
E = 320000
F_IN = 128
H = 256
C = 40
L = 8
ALPHA = 0.1
LAMDA = 0.5

NC = 2
NS = 16
CH = H // NC
K = 128
NPAD = 10112
EPAD = 327680
CHUNKS_SPMM = EPAD // K // NS
CHUNKS_DEG = EPAD // K // (NC * NS)

BN = 1000


def _deg_body(dst_hbm, zeros_hbm, out0, out1, dstv, onesv, degsh):
  c = lax.axis_index("c")
  s = lax.axis_index("s")
  one = jnp.ones((16,), jnp.float32)

  def fill_row(i, carry):
    def fill_col(g, carry2):
      onesv[i, pl.ds(g * 16, 16)] = one
      return carry2
    return lax.fori_loop(0, CH // 16, fill_col, carry)
  lax.fori_loop(0, K, fill_row, 0)

  @pl.when(s == 0)
  def _():
    pltpu.sync_copy(zeros_hbm, degsh)
  plsc.subcore_barrier()

  w = c * NS + s

  def chunk(j, carry):
    base = (w * CHUNKS_DEG + j) * K
    pltpu.sync_copy(dst_hbm.at[pl.ds(base, K)], dstv.at[0])
    pltpu.sync_copy(onesv, degsh.at[dstv.at[0]], add=True)
    return carry
  lax.fori_loop(0, CHUNKS_DEG, chunk, 0)
  plsc.subcore_barrier()

  @pl.when((s == 0) & (c == 0))
  def _():
    pltpu.sync_copy(degsh, out0)

  @pl.when((s == 0) & (c == 1))
  def _():
    pltpu.sync_copy(degsh, out1)


def _deg_call(dstp, zeros_big):
  mesh = plsc.VectorSubcoreMesh(core_axis_name="c", subcore_axis_name="s",
                                num_cores=NC, num_subcores=NS)
  f = pl.kernel(
      _deg_body,
      out_type=(jax.ShapeDtypeStruct((NPAD, CH), jnp.float32),
                jax.ShapeDtypeStruct((NPAD, CH), jnp.float32)),
      mesh=mesh,
      scratch_types=[
          pltpu.VMEM((1, K), jnp.int32),
          pltpu.VMEM((K, CH), jnp.float32),
          pltpu.VMEM_SHARED((NPAD, CH), jnp.float32),
      ],
  )
  return f(dstp, zeros_big)


def _spmm_body(g0, g1, src_hbm, dst_hbm, zeros_hbm, out0, out1,
               srcv, dstv, rows, aggsh, sem):
  c = lax.axis_index("c")
  s = lax.axis_index("s")

  @pl.when(s == 0)
  def _():
    pltpu.sync_copy(zeros_hbm, aggsh)
  plsc.subcore_barrier()

  def chunk(j, carry):
    base = (s * CHUNKS_SPMM + j) * K
    pltpu.sync_copy(src_hbm.at[pl.ds(base, K)], srcv.at[0])
    pltpu.sync_copy(dst_hbm.at[pl.ds(base, K)], dstv.at[0])

    @pl.when(c == 0)
    def _():
      pltpu.async_copy(g0.at[srcv.at[0]], rows, sem).wait()

    @pl.when(c == 1)
    def _():
      pltpu.async_copy(g1.at[srcv.at[0]], rows, sem).wait()

    pltpu.sync_copy(rows, aggsh.at[dstv.at[0]], add=True)
    return carry
  lax.fori_loop(0, CHUNKS_SPMM, chunk, 0)
  plsc.subcore_barrier()

  @pl.when((s == 0) & (c == 0))
  def _():
    pltpu.sync_copy(aggsh, out0)

  @pl.when((s == 0) & (c == 1))
  def _():
    pltpu.sync_copy(aggsh, out1)


def _spmm_call(g0, g1, srcp, dstp, zeros_big):
  mesh = plsc.VectorSubcoreMesh(core_axis_name="c", subcore_axis_name="s",
                                num_cores=NC, num_subcores=NS)
  f = pl.kernel(
      _spmm_body,
      out_type=(jax.ShapeDtypeStruct((NPAD, CH), jnp.float32),
                jax.ShapeDtypeStruct((NPAD, CH), jnp.float32)),
      mesh=mesh,
      scratch_types=[
          pltpu.VMEM((1, K), jnp.int32),
          pltpu.VMEM((1, K), jnp.int32),
          pltpu.VMEM((K, CH), jnp.float32),
          pltpu.VMEM_SHARED((NPAD, CH), jnp.float32),
          pltpu.SemaphoreType.DMA,
      ],
  )
  return f(g0, g1, srcp, dstp, zeros_big)


def _fc0_body(feat, w, b, degp0, degp1, h0_o, g0_o, g1_o, dinv_o):
  deg = degp0[:, 0:1] + degp1[:, 0:1]
  dinv = lax.rsqrt(jnp.maximum(deg, 1.0))
  h0 = jnp.maximum(
      jnp.dot(feat[...], w[...], preferred_element_type=jnp.float32) + b[...],
      0.0)
  h0_o[...] = h0
  gs = h0 * dinv
  g0_o[...] = gs[:, :CH]
  g1_o[...] = gs[:, CH:]
  dinv_o[...] = dinv


def _fc0_call(features, w0t, b0, degp0, degp1):
  grid = N // BN
  return pl.pallas_call(
      _fc0_body,
      grid=(grid,),
      in_specs=[
          pl.BlockSpec((BN, F_IN), lambda i: (i, 0)),
          pl.BlockSpec((F_IN, H), lambda i: (0, 0)),
          pl.BlockSpec((1, H), lambda i: (0, 0)),
          pl.BlockSpec((BN, CH), lambda i: (i, 0)),
          pl.BlockSpec((BN, CH), lambda i: (i, 0)),
      ],
      out_specs=[
          pl.BlockSpec((BN, H), lambda i: (i, 0)),
          pl.BlockSpec((BN, CH), lambda i: (i, 0)),
          pl.BlockSpec((BN, CH), lambda i: (i, 0)),
          pl.BlockSpec((BN, 1), lambda i: (i, 0)),
      ],
      out_shape=[
          jax.ShapeDtypeStruct((N, H), jnp.float32),
          jax.ShapeDtypeStruct((N, CH), jnp.float32),
          jax.ShapeDtypeStruct((N, CH), jnp.float32),
          jax.ShapeDtypeStruct((N, 1), jnp.float32),
      ],
  )(features, w0t, b0, degp0, degp1)


def _layer_body(agg0, agg1, h0, dinv, w, o0, o1, *, beta, scale_out):
  dv = dinv[...]
  agg = jnp.concatenate([agg0[...], agg1[...]], axis=-1) * dv
  hh = (1.0 - ALPHA) * agg + ALPHA * h0[...]
  z = (1.0 - beta) * hh + beta * jnp.dot(
      hh, w[...], preferred_element_type=jnp.float32)
  h = jnp.maximum(z, 0.0)
  if scale_out:
    h = h * dv
  o0[...] = h[:, :CH]
  o1[...] = h[:, CH:]


def _layer_call(agg0, agg1, h0, dinv, w, beta, scale_out):
  grid = N // BN
  return pl.pallas_call(
      functools.partial(_layer_body, beta=beta, scale_out=scale_out),
      grid=(grid,),
      in_specs=[
          pl.BlockSpec((BN, CH), lambda i: (i, 0)),
          pl.BlockSpec((BN, CH), lambda i: (i, 0)),
          pl.BlockSpec((BN, H), lambda i: (i, 0)),
          pl.BlockSpec((BN, 1), lambda i: (i, 0)),
          pl.BlockSpec((H, H), lambda i: (0, 0)),
      ],
      out_specs=[
          pl.BlockSpec((BN, CH), lambda i: (i, 0)),
          pl.BlockSpec((BN, CH), lambda i: (i, 0)),
      ],
      out_shape=[
          jax.ShapeDtypeStruct((N, CH), jnp.float32),
          jax.ShapeDtypeStruct((N, CH), jnp.float32),
      ],
  )(agg0, agg1, h0, dinv, w)


def _fc1_body(hl, hr, w, b, out):
  hcat = jnp.concatenate([hl[...], hr[...]], axis=-1)
  out[...] = jnp.dot(hcat, w[...], preferred_element_type=jnp.float32) + b[...]


def _fc1_call(hl, hr, w1t, b1):
  grid = N // BN
  return pl.pallas_call(
      _fc1_body,
      grid=(grid,),
      in_specs=[
          pl.BlockSpec((BN, CH), lambda i: (i, 0)),
          pl.BlockSpec((BN, CH), lambda i: (i, 0)),
          pl.BlockSpec((H, 128), lambda i: (0, 0)),
          pl.BlockSpec((1, 128), lambda i: (0, 0)),
      ],
      out_specs=pl.BlockSpec((BN, 128), lambda i: (i, 0)),
      out_shape=jax.ShapeDtypeStruct((N, 128), jnp.float32),
  )(hl, hr, w1t, b1)


def kernel(features, graph, fc0_W, fc0_b, conv_W, fc1_W, fc1_b):
  src = graph[0].astype(jnp.int32)
  dst = graph[1].astype(jnp.int32)
  pad = EPAD - E
  srcp = jnp.concatenate([src, jnp.zeros((pad,), jnp.int32)])
  dstp = jnp.concatenate([dst, jnp.full((pad,), N, jnp.int32)])
  zeros_big = jnp.zeros((NPAD, CH), jnp.float32)

  degp0, degp1 = _deg_call(dstp, zeros_big)

  w0t = fc0_W.T
  b0 = fc0_b.reshape(1, H)
  h0, g0, g1, dinv = _fc0_call(features, w0t, b0, degp0, degp1)

  for i in range(L):
    beta = math.log(LAMDA / (i + 1) + 1.0)
    agg0, agg1 = _spmm_call(g0, g1, srcp, dstp, zeros_big)
    g0, g1 = _layer_call(agg0, agg1, h0, dinv, conv_W[i],
                         beta, scale_out=(i < L - 1))

  w1t = jnp.zeros((H, 128), jnp.float32).at[:, :C].set(fc1_W.T)
  b1 = jnp.zeros((1, 128), jnp.float32).at[0, :C].set(fc1_b)
  out = _fc1_call(g0, g1, w1t, b1)
  return out[:, :C]


# speedup vs baseline: 5.2349x; 1.7977x over previous
import functools
import math

import jax
import jax.numpy as jnp
from jax import lax
from jax.experimental import pallas as pl
from jax.experimental.pallas import tpu as pltpu
from jax.experimental.pallas import tpu_sc as plsc

N = 10000
E = 320000
F_IN = 128
H = 256
C = 40
L = 8
ALPHA = 0.1
LAMDA = 0.5

NC = 2
NS = 16
CH = H // NC
K = 128
KS = 64
NPAD = 10112
EPAD = 327680
IDXPAD = 256
CHUNKS_SPMM = EPAD // KS // NS
CHUNKS_DEG = EPAD // K // (NC * NS)
NB = 4
NI = 8

BN = 1000


def _deg_body(dst_hbm, zeros_hbm, out0, out1, dstv, onesv, degsh):
  c = lax.axis_index("c")
  s = lax.axis_index("s")
  one = jnp.ones((16,), jnp.float32)

  def fill_row(i, carry):
    def fill_col(g, carry2):
      onesv[i, pl.ds(g * 16, 16)] = one
      return carry2
    return lax.fori_loop(0, CH // 16, fill_col, carry)
  lax.fori_loop(0, K, fill_row, 0)

  @pl.when(s == 0)
  def _():
    pltpu.sync_copy(zeros_hbm, degsh)
  plsc.subcore_barrier()

  w = c * NS + s

  def chunk(j, carry):
    base = (w * CHUNKS_DEG + j) * K
    pltpu.sync_copy(dst_hbm.at[pl.ds(base, K)], dstv.at[0])
    pltpu.sync_copy(onesv, degsh.at[dstv.at[0]], add=True)
    return carry
  lax.fori_loop(0, CHUNKS_DEG, chunk, 0)
  plsc.subcore_barrier()

  @pl.when((s == 0) & (c == 0))
  def _():
    pltpu.sync_copy(degsh, out0)

  @pl.when((s == 0) & (c == 1))
  def _():
    pltpu.sync_copy(degsh, out1)


def _deg_call(dstp, zeros_big):
  mesh = plsc.VectorSubcoreMesh(core_axis_name="c", subcore_axis_name="s",
                                num_cores=NC, num_subcores=NS)
  f = pl.kernel(
      _deg_body,
      out_type=(jax.ShapeDtypeStruct((NPAD, CH), jnp.float32),
                jax.ShapeDtypeStruct((NPAD, CH), jnp.float32)),
      mesh=mesh,
      scratch_types=[
          pltpu.VMEM((1, K), jnp.int32),
          pltpu.VMEM((K, CH), jnp.float32),
          pltpu.VMEM_SHARED((NPAD, CH), jnp.float32),
      ],
  )
  return f(dstp, zeros_big)


def _spmm_body(g0, g1, src_hbm, dst_hbm, zeros_hbm, out0, out1,
               idxv, rows, aggsh, isem, gsem, ssem):
  c = lax.axis_index("c")
  s = lax.axis_index("s")
  nchunks = CHUNKS_SPMM

  @pl.when(s == 0)
  def _():
    pltpu.sync_copy(zeros_hbm, aggsh)
  plsc.subcore_barrier()

  base0 = s * nchunks

  def issue_idx(j, bi):
    base = (base0 + j) * KS
    pltpu.async_copy(src_hbm.at[pl.ds(base, KS)], idxv[bi].at[0], isem[bi])
    pltpu.async_copy(dst_hbm.at[pl.ds(base, KS)], idxv[bi].at[1], isem[bi])

  def wait_idx(bi):
    pltpu.make_async_copy(src_hbm.at[pl.ds(0, KS)], idxv[bi].at[0],
                          isem[bi]).wait()
    pltpu.make_async_copy(dst_hbm.at[pl.ds(0, KS)], idxv[bi].at[1],
                          isem[bi]).wait()

  def issue_gather(bi, b):
    @pl.when(c == 0)
    def _():
      pltpu.async_copy(g0.at[idxv[bi].at[0]], rows[b], gsem[b])

    @pl.when(c == 1)
    def _():
      pltpu.async_copy(g1.at[idxv[bi].at[0]], rows[b], gsem[b])

  def wait_gather(b):
    pltpu.make_async_copy(g0.at[idxv[0].at[0]], rows[b], gsem[b]).wait()

  def issue_scatter(bi, b):
    pltpu.async_copy(rows[b], aggsh.at[idxv[bi].at[1]], ssem[b], add=True)

  def wait_scatter(b):
    pltpu.make_async_copy(rows[b], aggsh.at[idxv[0].at[1]], ssem[b]).wait()

  for jj in range(NB):
    issue_idx(jj, jj)
  wait_idx(0)
  issue_gather(0, 0)

  def octet(q, carry):
    for u in range(NI):
      j = q * NI + u
      b = u % NB
      issue_idx(j + NB, (u + NB) % NI)
      bn = (b + 1) % NB

      @pl.when(j + 1 < nchunks)
      def _():
        @pl.when(j + 1 >= NB)
        def _():
          wait_scatter(bn)
        wait_idx((u + 1) % NI)
        issue_gather((u + 1) % NI, bn)

      wait_gather(b)
      issue_scatter(u, b)
    return carry
  lax.fori_loop(0, nchunks // NI, octet, 0)

  for b in (1, 2, 3):
    wait_scatter(b)
  for bi in range(NB):
    wait_idx(bi)
  plsc.subcore_barrier()

  @pl.when((s == 0) & (c == 0))
  def _():
    pltpu.sync_copy(aggsh, out0)

  @pl.when((s == 0) & (c == 1))
  def _():
    pltpu.sync_copy(aggsh, out1)


def _spmm_call(g0, g1, srcp, dstp, zeros_big):
  mesh = plsc.VectorSubcoreMesh(core_axis_name="c", subcore_axis_name="s",
                                num_cores=NC, num_subcores=NS)
  f = pl.kernel(
      _spmm_body,
      out_type=(jax.ShapeDtypeStruct((NPAD, CH), jnp.float32),
                jax.ShapeDtypeStruct((NPAD, CH), jnp.float32)),
      mesh=mesh,
      scratch_types=[
          [pltpu.VMEM((2, KS), jnp.int32) for _ in range(NI)],
          [pltpu.VMEM((KS, CH), jnp.float32) for _ in range(NB)],
          pltpu.VMEM_SHARED((NPAD, CH), jnp.float32),
          [pltpu.SemaphoreType.DMA for _ in range(NI)],
          [pltpu.SemaphoreType.DMA for _ in range(NB)],
          [pltpu.SemaphoreType.DMA for _ in range(NB)],
      ],
  )
  return f(g0, g1, srcp, dstp, zeros_big)


def _fc0_body(feat, w, b, degp0, degp1, h0_o, g0_o, g1_o, dinv_o):
  deg = degp0[:, 0:1] + degp1[:, 0:1]
  dinv = lax.rsqrt(jnp.maximum(deg, 1.0))
  h0 = jnp.maximum(
      jnp.dot(feat[...], w[...], preferred_element_type=jnp.float32) + b[...],
      0.0)
  h0_o[...] = h0
  gs = h0 * dinv
  g0_o[...] = gs[:, :CH]
  g1_o[...] = gs[:, CH:]
  dinv_o[...] = dinv


def _fc0_call(features, w0t, b0, degp0, degp1):
  grid = N // BN
  return pl.pallas_call(
      _fc0_body,
      grid=(grid,),
      in_specs=[
          pl.BlockSpec((BN, F_IN), lambda i: (i, 0)),
          pl.BlockSpec((F_IN, H), lambda i: (0, 0)),
          pl.BlockSpec((1, H), lambda i: (0, 0)),
          pl.BlockSpec((BN, CH), lambda i: (i, 0)),
          pl.BlockSpec((BN, CH), lambda i: (i, 0)),
      ],
      out_specs=[
          pl.BlockSpec((BN, H), lambda i: (i, 0)),
          pl.BlockSpec((BN, CH), lambda i: (i, 0)),
          pl.BlockSpec((BN, CH), lambda i: (i, 0)),
          pl.BlockSpec((BN, 1), lambda i: (i, 0)),
      ],
      out_shape=[
          jax.ShapeDtypeStruct((N, H), jnp.float32),
          jax.ShapeDtypeStruct((N, CH), jnp.float32),
          jax.ShapeDtypeStruct((N, CH), jnp.float32),
          jax.ShapeDtypeStruct((N, 1), jnp.float32),
      ],
  )(features, w0t, b0, degp0, degp1)


def _layer_body(agg0, agg1, h0, dinv, w, o0, o1, *, beta, scale_out):
  dv = dinv[...]
  agg = jnp.concatenate([agg0[...], agg1[...]], axis=-1) * dv
  hh = (1.0 - ALPHA) * agg + ALPHA * h0[...]
  z = (1.0 - beta) * hh + beta * jnp.dot(
      hh, w[...], preferred_element_type=jnp.float32)
  h = jnp.maximum(z, 0.0)
  if scale_out:
    h = h * dv
  o0[...] = h[:, :CH]
  o1[...] = h[:, CH:]


def _layer_call(agg0, agg1, h0, dinv, w, beta, scale_out):
  grid = N // BN
  return pl.pallas_call(
      functools.partial(_layer_body, beta=beta, scale_out=scale_out),
      grid=(grid,),
      in_specs=[
          pl.BlockSpec((BN, CH), lambda i: (i, 0)),
          pl.BlockSpec((BN, CH), lambda i: (i, 0)),
          pl.BlockSpec((BN, H), lambda i: (i, 0)),
          pl.BlockSpec((BN, 1), lambda i: (i, 0)),
          pl.BlockSpec((H, H), lambda i: (0, 0)),
      ],
      out_specs=[
          pl.BlockSpec((BN, CH), lambda i: (i, 0)),
          pl.BlockSpec((BN, CH), lambda i: (i, 0)),
      ],
      out_shape=[
          jax.ShapeDtypeStruct((N, CH), jnp.float32),
          jax.ShapeDtypeStruct((N, CH), jnp.float32),
      ],
  )(agg0, agg1, h0, dinv, w)


def _fc1_body(hl, hr, w, b, out):
  hcat = jnp.concatenate([hl[...], hr[...]], axis=-1)
  out[...] = jnp.dot(hcat, w[...], preferred_element_type=jnp.float32) + b[...]


def _fc1_call(hl, hr, w1t, b1):
  grid = N // BN
  return pl.pallas_call(
      _fc1_body,
      grid=(grid,),
      in_specs=[
          pl.BlockSpec((BN, CH), lambda i: (i, 0)),
          pl.BlockSpec((BN, CH), lambda i: (i, 0)),
          pl.BlockSpec((H, 128), lambda i: (0, 0)),
          pl.BlockSpec((1, 128), lambda i: (0, 0)),
      ],
      out_specs=pl.BlockSpec((BN, 128), lambda i: (i, 0)),
      out_shape=jax.ShapeDtypeStruct((N, 128), jnp.float32),
  )(hl, hr, w1t, b1)


def kernel(features, graph, fc0_W, fc0_b, conv_W, fc1_W, fc1_b):
  src = graph[0].astype(jnp.int32)
  dst = graph[1].astype(jnp.int32)
  pad = EPAD - E
  srcp = jnp.concatenate(
      [src, jnp.zeros((pad + IDXPAD,), jnp.int32)])
  dstp = jnp.concatenate(
      [dst, jnp.full((pad,), N, jnp.int32), jnp.zeros((IDXPAD,), jnp.int32)])
  zeros_big = jnp.zeros((NPAD, CH), jnp.float32)

  degp0, degp1 = _deg_call(dstp, zeros_big)

  w0t = fc0_W.T
  b0 = fc0_b.reshape(1, H)
  h0, g0, g1, dinv = _fc0_call(features, w0t, b0, degp0, degp1)

  for i in range(L):
    beta = math.log(LAMDA / (i + 1) + 1.0)
    agg0, agg1 = _spmm_call(g0, g1, srcp, dstp, zeros_big)
    g0, g1 = _layer_call(agg0, agg1, h0, dinv, conv_W[i],
                         beta, scale_out=(i < L - 1))

  w1t = jnp.zeros((H, 128), jnp.float32).at[:, :C].set(fc1_W.T)
  b1 = jnp.zeros((1, 128), jnp.float32).at[0, :C].set(fc1_b)
  out = _fc1_call(g0, g1, w1t, b1)
  return out[:, :C]


# speedup vs baseline: 5.4874x; 1.0482x over previous
import functools
import math

import jax
import jax.numpy as jnp
from jax import lax
from jax.experimental import pallas as pl
from jax.experimental.pallas import tpu as pltpu
from jax.experimental.pallas import tpu_sc as plsc

N = 10000
E = 320000
F_IN = 128
H = 256
C = 40
L = 8
ALPHA = 0.1
LAMDA = 0.5

NC = 2
NS = 16
CH = H // NC
K = 128
KS = 64
NPAD = 10112
EPAD = 327680
IDXPAD = 256
CHUNKS_SPMM = EPAD // KS // NS
CHUNKS_DEG = EPAD // K // (NC * NS)
NB = 4
NI = 8

BN = 1000


def _deg_body(dst_hbm, zeros_hbm, out0, out1, dstv, onesv, degsh):
  c = lax.axis_index("c")
  s = lax.axis_index("s")
  one = jnp.ones((16,), jnp.float32)

  def fill_row(i, carry):
    def fill_col(g, carry2):
      onesv[i, pl.ds(g * 16, 16)] = one
      return carry2
    return lax.fori_loop(0, CH // 16, fill_col, carry)
  lax.fori_loop(0, K, fill_row, 0)

  @pl.when(s == 0)
  def _():
    pltpu.sync_copy(zeros_hbm, degsh)
  plsc.subcore_barrier()

  w = c * NS + s

  def chunk(j, carry):
    base = (w * CHUNKS_DEG + j) * K
    pltpu.sync_copy(dst_hbm.at[pl.ds(base, K)], dstv.at[0])
    pltpu.sync_copy(onesv, degsh.at[dstv.at[0]], add=True)
    return carry
  lax.fori_loop(0, CHUNKS_DEG, chunk, 0)
  plsc.subcore_barrier()

  @pl.when((s == 0) & (c == 0))
  def _():
    pltpu.sync_copy(degsh, out0)

  @pl.when((s == 0) & (c == 1))
  def _():
    pltpu.sync_copy(degsh, out1)


def _deg_call(dstp, zeros_big):
  mesh = plsc.VectorSubcoreMesh(core_axis_name="c", subcore_axis_name="s",
                                num_cores=NC, num_subcores=NS)
  f = pl.kernel(
      _deg_body,
      out_type=(jax.ShapeDtypeStruct((NPAD, CH), jnp.float32),
                jax.ShapeDtypeStruct((NPAD, CH), jnp.float32)),
      mesh=mesh,
      scratch_types=[
          pltpu.VMEM((1, K), jnp.int32),
          pltpu.VMEM((K, CH), jnp.float32),
          pltpu.VMEM_SHARED((NPAD, CH), jnp.float32),
      ],
  )
  return f(dstp, zeros_big)


def _spmm_body(g0, g1, src_hbm, dst_hbm, zeros_hbm, out0, out1,
               idxv, rows, aggsh, isem, gsem, ssem):
  c = lax.axis_index("c")
  s = lax.axis_index("s")
  nchunks = CHUNKS_SPMM

  @pl.when(s == 0)
  def _():
    pltpu.sync_copy(zeros_hbm, aggsh)
  plsc.subcore_barrier()

  base0 = s * nchunks

  def issue_idx(j, bi):
    base = (base0 + j) * KS
    pltpu.async_copy(src_hbm.at[pl.ds(base, KS)], idxv[bi].at[0], isem[bi])
    pltpu.async_copy(dst_hbm.at[pl.ds(base, KS)], idxv[bi].at[1], isem[bi])

  def wait_idx(bi):
    pltpu.make_async_copy(src_hbm.at[pl.ds(0, KS)], idxv[bi].at[0],
                          isem[bi]).wait()
    pltpu.make_async_copy(dst_hbm.at[pl.ds(0, KS)], idxv[bi].at[1],
                          isem[bi]).wait()

  def issue_gather(bi, b):
    @pl.when(c == 0)
    def _():
      pltpu.async_copy(g0.at[idxv[bi].at[0]], rows[b], gsem[b])

    @pl.when(c == 1)
    def _():
      pltpu.async_copy(g1.at[idxv[bi].at[0]], rows[b], gsem[b])

  def wait_gather(b):
    pltpu.make_async_copy(g0.at[idxv[0].at[0]], rows[b], gsem[b]).wait()

  def issue_scatter(bi, b):
    pltpu.async_copy(rows[b], aggsh.at[idxv[bi].at[1]], ssem[b], add=True)

  def wait_scatter(b):
    pltpu.make_async_copy(rows[b], aggsh.at[idxv[0].at[1]], ssem[b]).wait()

  for jj in range(NB):
    issue_idx(jj, jj)
  wait_idx(0)
  issue_gather(0, 0)
  wait_idx(1)
  issue_gather(1, 1)

  def octet(q, carry):
    for u in range(NI):
      j = q * NI + u
      b = u % NB
      issue_idx(j + NB, (u + NB) % NI)
      bn = (b + 2) % NB

      @pl.when(j + 2 < nchunks)
      def _():
        @pl.when(j + 2 >= NB)
        def _():
          wait_scatter(bn)
        wait_idx((u + 2) % NI)
        issue_gather((u + 2) % NI, bn)

      wait_gather(b)
      issue_scatter(u, b)
    return carry
  lax.fori_loop(0, nchunks // NI, octet, 0)

  for b in range(NB):
    wait_scatter(b)
    wait_idx(b)
  plsc.subcore_barrier()

  @pl.when((s == 0) & (c == 0))
  def _():
    pltpu.sync_copy(aggsh, out0)

  @pl.when((s == 0) & (c == 1))
  def _():
    pltpu.sync_copy(aggsh, out1)


def _spmm_call(g0, g1, srcp, dstp, zeros_big):
  mesh = plsc.VectorSubcoreMesh(core_axis_name="c", subcore_axis_name="s",
                                num_cores=NC, num_subcores=NS)
  f = pl.kernel(
      _spmm_body,
      out_type=(jax.ShapeDtypeStruct((NPAD, CH), jnp.float32),
                jax.ShapeDtypeStruct((NPAD, CH), jnp.float32)),
      mesh=mesh,
      scratch_types=[
          [pltpu.VMEM((2, KS), jnp.int32) for _ in range(NI)],
          [pltpu.VMEM((KS, CH), jnp.float32) for _ in range(NB)],
          pltpu.VMEM_SHARED((NPAD, CH), jnp.float32),
          [pltpu.SemaphoreType.DMA for _ in range(NI)],
          [pltpu.SemaphoreType.DMA for _ in range(NB)],
          [pltpu.SemaphoreType.DMA for _ in range(NB)],
      ],
  )
  return f(g0, g1, srcp, dstp, zeros_big)


def _fc0_body(feat, w, b, degp0, degp1, h0_o, g0_o, g1_o, dinv_o):
  deg = degp0[:, 0:1] + degp1[:, 0:1]
  dinv = lax.rsqrt(jnp.maximum(deg, 1.0))
  h0 = jnp.maximum(
      jnp.dot(feat[...], w[...], preferred_element_type=jnp.float32) + b[...],
      0.0)
  h0_o[...] = h0
  gs = h0 * dinv
  g0_o[...] = gs[:, :CH]
  g1_o[...] = gs[:, CH:]
  dinv_o[...] = dinv


def _fc0_call(features, w0t, b0, degp0, degp1):
  grid = N // BN
  return pl.pallas_call(
      _fc0_body,
      grid=(grid,),
      in_specs=[
          pl.BlockSpec((BN, F_IN), lambda i: (i, 0)),
          pl.BlockSpec((F_IN, H), lambda i: (0, 0)),
          pl.BlockSpec((1, H), lambda i: (0, 0)),
          pl.BlockSpec((BN, CH), lambda i: (i, 0)),
          pl.BlockSpec((BN, CH), lambda i: (i, 0)),
      ],
      out_specs=[
          pl.BlockSpec((BN, H), lambda i: (i, 0)),
          pl.BlockSpec((BN, CH), lambda i: (i, 0)),
          pl.BlockSpec((BN, CH), lambda i: (i, 0)),
          pl.BlockSpec((BN, 1), lambda i: (i, 0)),
      ],
      out_shape=[
          jax.ShapeDtypeStruct((N, H), jnp.float32),
          jax.ShapeDtypeStruct((N, CH), jnp.float32),
          jax.ShapeDtypeStruct((N, CH), jnp.float32),
          jax.ShapeDtypeStruct((N, 1), jnp.float32),
      ],
  )(features, w0t, b0, degp0, degp1)


def _layer_body(agg0, agg1, h0, dinv, w, o0, o1, *, beta, scale_out):
  dv = dinv[...]
  agg = jnp.concatenate([agg0[...], agg1[...]], axis=-1) * dv
  hh = (1.0 - ALPHA) * agg + ALPHA * h0[...]
  z = (1.0 - beta) * hh + beta * jnp.dot(
      hh, w[...], preferred_element_type=jnp.float32)
  h = jnp.maximum(z, 0.0)
  if scale_out:
    h = h * dv
  o0[...] = h[:, :CH]
  o1[...] = h[:, CH:]


def _layer_call(agg0, agg1, h0, dinv, w, beta, scale_out):
  grid = N // BN
  return pl.pallas_call(
      functools.partial(_layer_body, beta=beta, scale_out=scale_out),
      grid=(grid,),
      in_specs=[
          pl.BlockSpec((BN, CH), lambda i: (i, 0)),
          pl.BlockSpec((BN, CH), lambda i: (i, 0)),
          pl.BlockSpec((BN, H), lambda i: (i, 0)),
          pl.BlockSpec((BN, 1), lambda i: (i, 0)),
          pl.BlockSpec((H, H), lambda i: (0, 0)),
      ],
      out_specs=[
          pl.BlockSpec((BN, CH), lambda i: (i, 0)),
          pl.BlockSpec((BN, CH), lambda i: (i, 0)),
      ],
      out_shape=[
          jax.ShapeDtypeStruct((N, CH), jnp.float32),
          jax.ShapeDtypeStruct((N, CH), jnp.float32),
      ],
  )(agg0, agg1, h0, dinv, w)


def _fc1_body(hl, hr, w, b, out):
  hcat = jnp.concatenate([hl[...], hr[...]], axis=-1)
  out[...] = jnp.dot(hcat, w[...], preferred_element_type=jnp.float32) + b[...]


def _fc1_call(hl, hr, w1t, b1):
  grid = N // BN
  return pl.pallas_call(
      _fc1_body,
      grid=(grid,),
      in_specs=[
          pl.BlockSpec((BN, CH), lambda i: (i, 0)),
          pl.BlockSpec((BN, CH), lambda i: (i, 0)),
          pl.BlockSpec((H, 128), lambda i: (0, 0)),
          pl.BlockSpec((1, 128), lambda i: (0, 0)),
      ],
      out_specs=pl.BlockSpec((BN, 128), lambda i: (i, 0)),
      out_shape=jax.ShapeDtypeStruct((N, 128), jnp.float32),
  )(hl, hr, w1t, b1)


def kernel(features, graph, fc0_W, fc0_b, conv_W, fc1_W, fc1_b):
  src = graph[0].astype(jnp.int32)
  dst = graph[1].astype(jnp.int32)
  pad = EPAD - E
  srcp = jnp.concatenate(
      [src, jnp.zeros((pad + IDXPAD,), jnp.int32)])
  dstp = jnp.concatenate(
      [dst, jnp.full((pad,), N, jnp.int32), jnp.zeros((IDXPAD,), jnp.int32)])
  zeros_big = jnp.zeros((NPAD, CH), jnp.float32)

  degp0, degp1 = _deg_call(dstp, zeros_big)

  w0t = fc0_W.T
  b0 = fc0_b.reshape(1, H)
  h0, g0, g1, dinv = _fc0_call(features, w0t, b0, degp0, degp1)

  for i in range(L):
    beta = math.log(LAMDA / (i + 1) + 1.0)
    agg0, agg1 = _spmm_call(g0, g1, srcp, dstp, zeros_big)
    g0, g1 = _layer_call(agg0, agg1, h0, dinv, conv_W[i],
                         beta, scale_out=(i < L - 1))

  w1t = jnp.zeros((H, 128), jnp.float32).at[:, :C].set(fc1_W.T)
  b1 = jnp.zeros((1, 128), jnp.float32).at[0, :C].set(fc1_b)
  out = _fc1_call(g0, g1, w1t, b1)
  return out[:, :C]


# speedup vs baseline: 5.4921x; 1.0009x over previous
import functools
import math

import jax
import jax.numpy as jnp
from jax import lax
from jax.experimental import pallas as pl
from jax.experimental.pallas import tpu as pltpu
from jax.experimental.pallas import tpu_sc as plsc

N = 10000
E = 320000
F_IN = 128
H = 256
C = 40
L = 8
ALPHA = 0.1
LAMDA = 0.5

NC = 2
NS = 16
CH = H // NC
K = 128
KS = 80
NPAD = 10112
EPAD = 327680
IDXPAD = 4 * KS
CHUNKS_SPMM = EPAD // KS // NS
CHUNKS_DEG = EPAD // K // (NC * NS)
NB = 4
NI = 8

BN = 1000


def _deg_body(dst_hbm, zeros_hbm, out0, out1, dstv, onesv, degsh):
  c = lax.axis_index("c")
  s = lax.axis_index("s")
  one = jnp.ones((16,), jnp.float32)

  def fill_row(i, carry):
    def fill_col(g, carry2):
      onesv[i, pl.ds(g * 16, 16)] = one
      return carry2
    return lax.fori_loop(0, CH // 16, fill_col, carry)
  lax.fori_loop(0, K, fill_row, 0)

  @pl.when(s == 0)
  def _():
    pltpu.sync_copy(zeros_hbm, degsh)
  plsc.subcore_barrier()

  w = c * NS + s

  def chunk(j, carry):
    base = (w * CHUNKS_DEG + j) * K
    pltpu.sync_copy(dst_hbm.at[pl.ds(base, K)], dstv.at[0])
    pltpu.sync_copy(onesv, degsh.at[dstv.at[0]], add=True)
    return carry
  lax.fori_loop(0, CHUNKS_DEG, chunk, 0)
  plsc.subcore_barrier()

  @pl.when((s == 0) & (c == 0))
  def _():
    pltpu.sync_copy(degsh, out0)

  @pl.when((s == 0) & (c == 1))
  def _():
    pltpu.sync_copy(degsh, out1)


def _deg_call(dstp, zeros_big):
  mesh = plsc.VectorSubcoreMesh(core_axis_name="c", subcore_axis_name="s",
                                num_cores=NC, num_subcores=NS)
  f = pl.kernel(
      _deg_body,
      out_type=(jax.ShapeDtypeStruct((NPAD, CH), jnp.float32),
                jax.ShapeDtypeStruct((NPAD, CH), jnp.float32)),
      mesh=mesh,
      scratch_types=[
          pltpu.VMEM((1, K), jnp.int32),
          pltpu.VMEM((K, CH), jnp.float32),
          pltpu.VMEM_SHARED((NPAD, CH), jnp.float32),
      ],
  )
  return f(dstp, zeros_big)


def _spmm_body(g0, g1, src_hbm, dst_hbm, zeros_hbm, out0, out1,
               idxv, rows, aggsh, isem, gsem, ssem):
  c = lax.axis_index("c")
  s = lax.axis_index("s")
  nchunks = CHUNKS_SPMM

  @pl.when(s == 0)
  def _():
    pltpu.sync_copy(zeros_hbm, aggsh)
  plsc.subcore_barrier()

  base0 = s * nchunks

  def issue_idx(j, bi):
    base = (base0 + j) * KS
    pltpu.async_copy(src_hbm.at[pl.ds(base, KS)], idxv[bi].at[0], isem[bi])
    pltpu.async_copy(dst_hbm.at[pl.ds(base, KS)], idxv[bi].at[1], isem[bi])

  def wait_idx(bi):
    pltpu.make_async_copy(src_hbm.at[pl.ds(0, KS)], idxv[bi].at[0],
                          isem[bi]).wait()
    pltpu.make_async_copy(dst_hbm.at[pl.ds(0, KS)], idxv[bi].at[1],
                          isem[bi]).wait()

  def issue_gather(bi, b):
    @pl.when(c == 0)
    def _():
      pltpu.async_copy(g0.at[idxv[bi].at[0]], rows[b], gsem[b])

    @pl.when(c == 1)
    def _():
      pltpu.async_copy(g1.at[idxv[bi].at[0]], rows[b], gsem[b])

  def wait_gather(b):
    pltpu.make_async_copy(g0.at[idxv[0].at[0]], rows[b], gsem[b]).wait()

  def issue_scatter(bi, b):
    pltpu.async_copy(rows[b], aggsh.at[idxv[bi].at[1]], ssem[b], add=True)

  def wait_scatter(b):
    pltpu.make_async_copy(rows[b], aggsh.at[idxv[0].at[1]], ssem[b]).wait()

  for jj in range(NB):
    issue_idx(jj, jj)
  wait_idx(0)
  issue_gather(0, 0)
  wait_idx(1)
  issue_gather(1, 1)

  def octet(q, carry):
    for u in range(NI):
      j = q * NI + u
      b = u % NB
      issue_idx(j + NB, (u + NB) % NI)
      bn = (b + 2) % NB

      @pl.when(j + 2 < nchunks)
      def _():
        @pl.when(j + 2 >= NB)
        def _():
          wait_scatter(bn)
        wait_idx((u + 2) % NI)
        issue_gather((u + 2) % NI, bn)

      wait_gather(b)
      issue_scatter(u, b)
    return carry
  lax.fori_loop(0, nchunks // NI, octet, 0)

  for b in range(NB):
    wait_scatter(b)
    wait_idx(b)
  plsc.subcore_barrier()

  @pl.when((s == 0) & (c == 0))
  def _():
    pltpu.sync_copy(aggsh, out0)

  @pl.when((s == 0) & (c == 1))
  def _():
    pltpu.sync_copy(aggsh, out1)


def _spmm_call(g0, g1, srcp, dstp, zeros_big):
  mesh = plsc.VectorSubcoreMesh(core_axis_name="c", subcore_axis_name="s",
                                num_cores=NC, num_subcores=NS)
  f = pl.kernel(
      _spmm_body,
      out_type=(jax.ShapeDtypeStruct((NPAD, CH), jnp.float32),
                jax.ShapeDtypeStruct((NPAD, CH), jnp.float32)),
      mesh=mesh,
      scratch_types=[
          [pltpu.VMEM((2, KS), jnp.int32) for _ in range(NI)],
          [pltpu.VMEM((KS, CH), jnp.float32) for _ in range(NB)],
          pltpu.VMEM_SHARED((NPAD, CH), jnp.float32),
          [pltpu.SemaphoreType.DMA for _ in range(NI)],
          [pltpu.SemaphoreType.DMA for _ in range(NB)],
          [pltpu.SemaphoreType.DMA for _ in range(NB)],
      ],
  )
  return f(g0, g1, srcp, dstp, zeros_big)


def _fc0_body(feat, w, b, degp0, degp1, h0_o, g0_o, g1_o, dinv_o):
  deg = degp0[:, 0:1] + degp1[:, 0:1]
  dinv = lax.rsqrt(jnp.maximum(deg, 1.0))
  h0 = jnp.maximum(
      jnp.dot(feat[...], w[...], preferred_element_type=jnp.float32) + b[...],
      0.0)
  h0_o[...] = h0
  gs = h0 * dinv
  g0_o[...] = gs[:, :CH]
  g1_o[...] = gs[:, CH:]
  dinv_o[...] = dinv


def _fc0_call(features, w0t, b0, degp0, degp1):
  grid = N // BN
  return pl.pallas_call(
      _fc0_body,
      grid=(grid,),
      in_specs=[
          pl.BlockSpec((BN, F_IN), lambda i: (i, 0)),
          pl.BlockSpec((F_IN, H), lambda i: (0, 0)),
          pl.BlockSpec((1, H), lambda i: (0, 0)),
          pl.BlockSpec((BN, CH), lambda i: (i, 0)),
          pl.BlockSpec((BN, CH), lambda i: (i, 0)),
      ],
      out_specs=[
          pl.BlockSpec((BN, H), lambda i: (i, 0)),
          pl.BlockSpec((BN, CH), lambda i: (i, 0)),
          pl.BlockSpec((BN, CH), lambda i: (i, 0)),
          pl.BlockSpec((BN, 1), lambda i: (i, 0)),
      ],
      out_shape=[
          jax.ShapeDtypeStruct((N, H), jnp.float32),
          jax.ShapeDtypeStruct((N, CH), jnp.float32),
          jax.ShapeDtypeStruct((N, CH), jnp.float32),
          jax.ShapeDtypeStruct((N, 1), jnp.float32),
      ],
  )(features, w0t, b0, degp0, degp1)


def _layer_body(agg0, agg1, h0, dinv, w, o0, o1, *, beta, scale_out):
  dv = dinv[...]
  agg = jnp.concatenate([agg0[...], agg1[...]], axis=-1) * dv
  hh = (1.0 - ALPHA) * agg + ALPHA * h0[...]
  z = (1.0 - beta) * hh + beta * jnp.dot(
      hh, w[...], preferred_element_type=jnp.float32)
  h = jnp.maximum(z, 0.0)
  if scale_out:
    h = h * dv
  o0[...] = h[:, :CH]
  o1[...] = h[:, CH:]


def _layer_call(agg0, agg1, h0, dinv, w, beta, scale_out):
  grid = N // BN
  return pl.pallas_call(
      functools.partial(_layer_body, beta=beta, scale_out=scale_out),
      grid=(grid,),
      in_specs=[
          pl.BlockSpec((BN, CH), lambda i: (i, 0)),
          pl.BlockSpec((BN, CH), lambda i: (i, 0)),
          pl.BlockSpec((BN, H), lambda i: (i, 0)),
          pl.BlockSpec((BN, 1), lambda i: (i, 0)),
          pl.BlockSpec((H, H), lambda i: (0, 0)),
      ],
      out_specs=[
          pl.BlockSpec((BN, CH), lambda i: (i, 0)),
          pl.BlockSpec((BN, CH), lambda i: (i, 0)),
      ],
      out_shape=[
          jax.ShapeDtypeStruct((N, CH), jnp.float32),
          jax.ShapeDtypeStruct((N, CH), jnp.float32),
      ],
  )(agg0, agg1, h0, dinv, w)


def _fc1_body(hl, hr, w, b, out):
  hcat = jnp.concatenate([hl[...], hr[...]], axis=-1)
  out[...] = jnp.dot(hcat, w[...], preferred_element_type=jnp.float32) + b[...]


def _fc1_call(hl, hr, w1t, b1):
  grid = N // BN
  return pl.pallas_call(
      _fc1_body,
      grid=(grid,),
      in_specs=[
          pl.BlockSpec((BN, CH), lambda i: (i, 0)),
          pl.BlockSpec((BN, CH), lambda i: (i, 0)),
          pl.BlockSpec((H, 128), lambda i: (0, 0)),
          pl.BlockSpec((1, 128), lambda i: (0, 0)),
      ],
      out_specs=pl.BlockSpec((BN, 128), lambda i: (i, 0)),
      out_shape=jax.ShapeDtypeStruct((N, 128), jnp.float32),
  )(hl, hr, w1t, b1)


def kernel(features, graph, fc0_W, fc0_b, conv_W, fc1_W, fc1_b):
  src = graph[0].astype(jnp.int32)
  dst = graph[1].astype(jnp.int32)
  pad = EPAD - E
  srcp = jnp.concatenate(
      [src, jnp.zeros((pad + IDXPAD,), jnp.int32)])
  dstp = jnp.concatenate(
      [dst, jnp.full((pad,), N, jnp.int32), jnp.zeros((IDXPAD,), jnp.int32)])
  zeros_big = jnp.zeros((NPAD, CH), jnp.float32)

  degp0, degp1 = _deg_call(dstp, zeros_big)

  w0t = fc0_W.T
  b0 = fc0_b.reshape(1, H)
  h0, g0, g1, dinv = _fc0_call(features, w0t, b0, degp0, degp1)

  for i in range(L):
    beta = math.log(LAMDA / (i + 1) + 1.0)
    agg0, agg1 = _spmm_call(g0, g1, srcp, dstp, zeros_big)
    g0, g1 = _layer_call(agg0, agg1, h0, dinv, conv_W[i],
                         beta, scale_out=(i < L - 1))

  w1t = jnp.zeros((H, 128), jnp.float32).at[:, :C].set(fc1_W.T)
  b1 = jnp.zeros((1, 128), jnp.float32).at[0, :C].set(fc1_b)
  out = _fc1_call(g0, g1, w1t, b1)
  return out[:, :C]


# speedup vs baseline: 5.5475x; 1.0101x over previous
import functools
import math

import jax
import jax.numpy as jnp
from jax import lax
from jax.experimental import pallas as pl
from jax.experimental.pallas import tpu as pltpu
from jax.experimental.pallas import tpu_sc as plsc

N = 10000
E = 320000
F_IN = 128
H = 256
C = 40
L = 8
ALPHA = 0.1
LAMDA = 0.5

NC = 2
NS = 16
CH = H // NC
K = 128
KS = 64
NPAD = 10112
EPAD = 327680
IDXPAD = 4 * KS
CHUNKS_SPMM = EPAD // KS // NS
CHUNKS_DEG = EPAD // K // (NC * NS)
NB = 5
NI = 10
GA = 3

BN = 1000


def _deg_body(dst_hbm, zeros_hbm, out0, out1, dstv, onesv, degsh):
  c = lax.axis_index("c")
  s = lax.axis_index("s")
  one = jnp.ones((16,), jnp.float32)

  def fill_row(i, carry):
    def fill_col(g, carry2):
      onesv[i, pl.ds(g * 16, 16)] = one
      return carry2
    return lax.fori_loop(0, CH // 16, fill_col, carry)
  lax.fori_loop(0, K, fill_row, 0)

  @pl.when(s == 0)
  def _():
    pltpu.sync_copy(zeros_hbm, degsh)
  plsc.subcore_barrier()

  w = c * NS + s

  def chunk(j, carry):
    base = (w * CHUNKS_DEG + j) * K
    pltpu.sync_copy(dst_hbm.at[pl.ds(base, K)], dstv.at[0])
    pltpu.sync_copy(onesv, degsh.at[dstv.at[0]], add=True)
    return carry
  lax.fori_loop(0, CHUNKS_DEG, chunk, 0)
  plsc.subcore_barrier()

  @pl.when((s == 0) & (c == 0))
  def _():
    pltpu.sync_copy(degsh, out0)

  @pl.when((s == 0) & (c == 1))
  def _():
    pltpu.sync_copy(degsh, out1)


def _deg_call(dstp, zeros_big):
  mesh = plsc.VectorSubcoreMesh(core_axis_name="c", subcore_axis_name="s",
                                num_cores=NC, num_subcores=NS)
  f = pl.kernel(
      _deg_body,
      out_type=(jax.ShapeDtypeStruct((NPAD, CH), jnp.float32),
                jax.ShapeDtypeStruct((NPAD, CH), jnp.float32)),
      mesh=mesh,
      scratch_types=[
          pltpu.VMEM((1, K), jnp.int32),
          pltpu.VMEM((K, CH), jnp.float32),
          pltpu.VMEM_SHARED((NPAD, CH), jnp.float32),
      ],
  )
  return f(dstp, zeros_big)


def _spmm_body(g0, g1, src_hbm, dst_hbm, zeros_hbm, out0, out1,
               idxv, rows, aggsh, isem, gsem, ssem):
  c = lax.axis_index("c")
  s = lax.axis_index("s")
  nchunks = CHUNKS_SPMM

  @pl.when(s == 0)
  def _():
    pltpu.sync_copy(zeros_hbm, aggsh)
  plsc.subcore_barrier()

  base0 = s * nchunks

  def issue_idx(j, bi):
    base = (base0 + j) * KS
    pltpu.async_copy(src_hbm.at[pl.ds(base, KS)], idxv[bi].at[0], isem[bi])
    pltpu.async_copy(dst_hbm.at[pl.ds(base, KS)], idxv[bi].at[1], isem[bi])

  def wait_idx(bi):
    pltpu.make_async_copy(src_hbm.at[pl.ds(0, KS)], idxv[bi].at[0],
                          isem[bi]).wait()
    pltpu.make_async_copy(dst_hbm.at[pl.ds(0, KS)], idxv[bi].at[1],
                          isem[bi]).wait()

  def issue_gather(bi, b):
    @pl.when(c == 0)
    def _():
      pltpu.async_copy(g0.at[idxv[bi].at[0]], rows[b], gsem[b])

    @pl.when(c == 1)
    def _():
      pltpu.async_copy(g1.at[idxv[bi].at[0]], rows[b], gsem[b])

  def wait_gather(b):
    pltpu.make_async_copy(g0.at[idxv[0].at[0]], rows[b], gsem[b]).wait()

  def issue_scatter(bi, b):
    pltpu.async_copy(rows[b], aggsh.at[idxv[bi].at[1]], ssem[b], add=True)

  def wait_scatter(b):
    pltpu.make_async_copy(rows[b], aggsh.at[idxv[0].at[1]], ssem[b]).wait()

  for jj in range(4):
    issue_idx(jj, jj)
  for jj in range(GA):
    wait_idx(jj)
    issue_gather(jj, jj)

  def group(q, carry):
    for u in range(NI):
      j = q * NI + u
      b = u % NB
      issue_idx(j + 4, (u + 4) % NI)
      bn = (b + GA) % NB

      @pl.when(j + GA < nchunks)
      def _():
        @pl.when(j + GA >= NB)
        def _():
          wait_scatter(bn)
        wait_idx((u + GA) % NI)
        issue_gather((u + GA) % NI, bn)

      wait_gather(b)
      issue_scatter(u, b)
    return carry
  lax.fori_loop(0, nchunks // NI, group, 0)

  for b in range(NB):
    wait_scatter(b)
  for bi in range(4):
    wait_idx(bi)
  plsc.subcore_barrier()

  @pl.when((s == 0) & (c == 0))
  def _():
    pltpu.sync_copy(aggsh, out0)

  @pl.when((s == 0) & (c == 1))
  def _():
    pltpu.sync_copy(aggsh, out1)


def _spmm_call(g0, g1, srcp, dstp, zeros_big):
  mesh = plsc.VectorSubcoreMesh(core_axis_name="c", subcore_axis_name="s",
                                num_cores=NC, num_subcores=NS)
  f = pl.kernel(
      _spmm_body,
      out_type=(jax.ShapeDtypeStruct((NPAD, CH), jnp.float32),
                jax.ShapeDtypeStruct((NPAD, CH), jnp.float32)),
      mesh=mesh,
      scratch_types=[
          [pltpu.VMEM((2, KS), jnp.int32) for _ in range(NI)],
          [pltpu.VMEM((KS, CH), jnp.float32) for _ in range(NB)],
          pltpu.VMEM_SHARED((NPAD, CH), jnp.float32),
          [pltpu.SemaphoreType.DMA for _ in range(NI)],
          [pltpu.SemaphoreType.DMA for _ in range(NB)],
          [pltpu.SemaphoreType.DMA for _ in range(NB)],
      ],
  )
  return f(g0, g1, srcp, dstp, zeros_big)


def _fc0_body(feat, w, b, degp0, degp1, h0_o, g0_o, g1_o, dinv_o):
  deg = degp0[:, 0:1] + degp1[:, 0:1]
  dinv = lax.rsqrt(jnp.maximum(deg, 1.0))
  h0 = jnp.maximum(
      jnp.dot(feat[...], w[...], preferred_element_type=jnp.float32) + b[...],
      0.0)
  h0_o[...] = h0
  gs = h0 * dinv
  g0_o[...] = gs[:, :CH]
  g1_o[...] = gs[:, CH:]
  dinv_o[...] = dinv


def _fc0_call(features, w0t, b0, degp0, degp1):
  grid = N // BN
  return pl.pallas_call(
      _fc0_body,
      grid=(grid,),
      in_specs=[
          pl.BlockSpec((BN, F_IN), lambda i: (i, 0)),
          pl.BlockSpec((F_IN, H), lambda i: (0, 0)),
          pl.BlockSpec((1, H), lambda i: (0, 0)),
          pl.BlockSpec((BN, CH), lambda i: (i, 0)),
          pl.BlockSpec((BN, CH), lambda i: (i, 0)),
      ],
      out_specs=[
          pl.BlockSpec((BN, H), lambda i: (i, 0)),
          pl.BlockSpec((BN, CH), lambda i: (i, 0)),
          pl.BlockSpec((BN, CH), lambda i: (i, 0)),
          pl.BlockSpec((BN, 1), lambda i: (i, 0)),
      ],
      out_shape=[
          jax.ShapeDtypeStruct((N, H), jnp.float32),
          jax.ShapeDtypeStruct((N, CH), jnp.float32),
          jax.ShapeDtypeStruct((N, CH), jnp.float32),
          jax.ShapeDtypeStruct((N, 1), jnp.float32),
      ],
  )(features, w0t, b0, degp0, degp1)


def _layer_body(agg0, agg1, h0, dinv, w, o0, o1, *, beta, scale_out):
  dv = dinv[...]
  agg = jnp.concatenate([agg0[...], agg1[...]], axis=-1) * dv
  hh = (1.0 - ALPHA) * agg + ALPHA * h0[...]
  z = (1.0 - beta) * hh + beta * jnp.dot(
      hh, w[...], preferred_element_type=jnp.float32)
  h = jnp.maximum(z, 0.0)
  if scale_out:
    h = h * dv
  o0[...] = h[:, :CH]
  o1[...] = h[:, CH:]


def _layer_call(agg0, agg1, h0, dinv, w, beta, scale_out):
  grid = N // BN
  return pl.pallas_call(
      functools.partial(_layer_body, beta=beta, scale_out=scale_out),
      grid=(grid,),
      in_specs=[
          pl.BlockSpec((BN, CH), lambda i: (i, 0)),
          pl.BlockSpec((BN, CH), lambda i: (i, 0)),
          pl.BlockSpec((BN, H), lambda i: (i, 0)),
          pl.BlockSpec((BN, 1), lambda i: (i, 0)),
          pl.BlockSpec((H, H), lambda i: (0, 0)),
      ],
      out_specs=[
          pl.BlockSpec((BN, CH), lambda i: (i, 0)),
          pl.BlockSpec((BN, CH), lambda i: (i, 0)),
      ],
      out_shape=[
          jax.ShapeDtypeStruct((N, CH), jnp.float32),
          jax.ShapeDtypeStruct((N, CH), jnp.float32),
      ],
  )(agg0, agg1, h0, dinv, w)


def _fc1_body(hl, hr, w, b, out):
  hcat = jnp.concatenate([hl[...], hr[...]], axis=-1)
  out[...] = jnp.dot(hcat, w[...], preferred_element_type=jnp.float32) + b[...]


def _fc1_call(hl, hr, w1t, b1):
  grid = N // BN
  return pl.pallas_call(
      _fc1_body,
      grid=(grid,),
      in_specs=[
          pl.BlockSpec((BN, CH), lambda i: (i, 0)),
          pl.BlockSpec((BN, CH), lambda i: (i, 0)),
          pl.BlockSpec((H, 128), lambda i: (0, 0)),
          pl.BlockSpec((1, 128), lambda i: (0, 0)),
      ],
      out_specs=pl.BlockSpec((BN, 128), lambda i: (i, 0)),
      out_shape=jax.ShapeDtypeStruct((N, 128), jnp.float32),
  )(hl, hr, w1t, b1)


def kernel(features, graph, fc0_W, fc0_b, conv_W, fc1_W, fc1_b):
  src = graph[0].astype(jnp.int32)
  dst = graph[1].astype(jnp.int32)
  pad = EPAD - E
  srcp = jnp.concatenate(
      [src, jnp.zeros((pad + IDXPAD,), jnp.int32)])
  dstp = jnp.concatenate(
      [dst, jnp.full((pad,), N, jnp.int32), jnp.zeros((IDXPAD,), jnp.int32)])
  zeros_big = jnp.zeros((NPAD, CH), jnp.float32)

  degp0, degp1 = _deg_call(dstp, zeros_big)

  w0t = fc0_W.T
  b0 = fc0_b.reshape(1, H)
  h0, g0, g1, dinv = _fc0_call(features, w0t, b0, degp0, degp1)

  for i in range(L):
    beta = math.log(LAMDA / (i + 1) + 1.0)
    agg0, agg1 = _spmm_call(g0, g1, srcp, dstp, zeros_big)
    g0, g1 = _layer_call(agg0, agg1, h0, dinv, conv_W[i],
                         beta, scale_out=(i < L - 1))

  w1t = jnp.zeros((H, 128), jnp.float32).at[:, :C].set(fc1_W.T)
  b1 = jnp.zeros((1, 128), jnp.float32).at[0, :C].set(fc1_b)
  out = _fc1_call(g0, g1, w1t, b1)
  return out[:, :C]


# speedup vs baseline: 5.5960x; 1.0087x over previous
import functools
import math

import jax
import jax.numpy as jnp
from jax import lax
from jax.experimental import pallas as pl
from jax.experimental.pallas import tpu as pltpu
from jax.experimental.pallas import tpu_sc as plsc

N = 10000
E = 320000
F_IN = 128
H = 256
C = 40
L = 8
ALPHA = 0.1
LAMDA = 0.5

NC = 2
NS = 16
CH = H // NC
K = 128
KS = 64
NPAD = 10112
EPAD = 327680
IDXPAD = 4 * K
CHUNKS_SPMM = EPAD // KS // NS
CHUNKS_DEG = EPAD // K // (NC * NS)
NB = 5
NI = 10
GA = 3

BN = 1000


def _deg_body(dst_hbm, zeros_hbm, out0, out1, dstv, onesv, degsh, isem, ssem):
  c = lax.axis_index("c")
  s = lax.axis_index("s")
  one = jnp.ones((16,), jnp.float32)

  def fill_row(i, carry):
    def fill_col(g, carry2):
      onesv[i, pl.ds(g * 16, 16)] = one
      return carry2
    return lax.fori_loop(0, CH // 16, fill_col, carry)
  lax.fori_loop(0, K, fill_row, 0)

  @pl.when(s == 0)
  def _():
    pltpu.sync_copy(zeros_hbm, degsh)
  plsc.subcore_barrier()

  w = c * NS + s

  def issue_idx(j, bi):
    base = (w * CHUNKS_DEG + j) * K
    pltpu.async_copy(dst_hbm.at[pl.ds(base, K)], dstv[bi].at[0], isem[bi])

  def wait_idx(bi):
    pltpu.make_async_copy(dst_hbm.at[pl.ds(0, K)], dstv[bi].at[0],
                          isem[bi]).wait()

  def issue_scatter(bi, b):
    pltpu.async_copy(onesv, degsh.at[dstv[bi].at[0]], ssem[b], add=True)

  def wait_scatter(b):
    pltpu.make_async_copy(onesv, degsh.at[dstv[0].at[0]], ssem[b]).wait()

  for jj in range(4):
    issue_idx(jj, jj)

  def group(q, carry):
    for u in range(8):
      j = q * 8 + u

      @pl.when(j >= 4)
      def _():
        wait_scatter(u % 4)
      issue_idx(j + 4, (u + 4) % 8)
      wait_idx(u)
      issue_scatter(u, u % 4)
    return carry
  lax.fori_loop(0, CHUNKS_DEG // 8, group, 0)

  for b in range(4):
    wait_scatter(b)
    wait_idx(b)
  plsc.subcore_barrier()

  @pl.when((s == 0) & (c == 0))
  def _():
    pltpu.sync_copy(degsh, out0)

  @pl.when((s == 0) & (c == 1))
  def _():
    pltpu.sync_copy(degsh, out1)


def _deg_call(dstp, zeros_big):
  mesh = plsc.VectorSubcoreMesh(core_axis_name="c", subcore_axis_name="s",
                                num_cores=NC, num_subcores=NS)
  f = pl.kernel(
      _deg_body,
      out_type=(jax.ShapeDtypeStruct((NPAD, CH), jnp.float32),
                jax.ShapeDtypeStruct((NPAD, CH), jnp.float32)),
      mesh=mesh,
      scratch_types=[
          [pltpu.VMEM((1, K), jnp.int32) for _ in range(8)],
          pltpu.VMEM((K, CH), jnp.float32),
          pltpu.VMEM_SHARED((NPAD, CH), jnp.float32),
          [pltpu.SemaphoreType.DMA for _ in range(8)],
          [pltpu.SemaphoreType.DMA for _ in range(4)],
      ],
  )
  return f(dstp, zeros_big)


def _spmm_body(g0, g1, src_hbm, dst_hbm, zeros_hbm, out0, out1,
               idxv, rows, aggsh, isem, gsem, ssem):
  c = lax.axis_index("c")
  s = lax.axis_index("s")
  nchunks = CHUNKS_SPMM

  @pl.when(s == 0)
  def _():
    pltpu.sync_copy(zeros_hbm, aggsh)
  plsc.subcore_barrier()

  base0 = s * nchunks

  def issue_idx(j, bi):
    base = (base0 + j) * KS
    pltpu.async_copy(src_hbm.at[pl.ds(base, KS)], idxv[bi].at[0], isem[bi])
    pltpu.async_copy(dst_hbm.at[pl.ds(base, KS)], idxv[bi].at[1], isem[bi])

  def wait_idx(bi):
    pltpu.make_async_copy(src_hbm.at[pl.ds(0, KS)], idxv[bi].at[0],
                          isem[bi]).wait()
    pltpu.make_async_copy(dst_hbm.at[pl.ds(0, KS)], idxv[bi].at[1],
                          isem[bi]).wait()

  def issue_gather(bi, b):
    @pl.when(c == 0)
    def _():
      pltpu.async_copy(g0.at[idxv[bi].at[0]], rows[b], gsem[b])

    @pl.when(c == 1)
    def _():
      pltpu.async_copy(g1.at[idxv[bi].at[0]], rows[b], gsem[b])

  def wait_gather(b):
    pltpu.make_async_copy(g0.at[idxv[0].at[0]], rows[b], gsem[b]).wait()

  def issue_scatter(bi, b):
    pltpu.async_copy(rows[b], aggsh.at[idxv[bi].at[1]], ssem[b], add=True)

  def wait_scatter(b):
    pltpu.make_async_copy(rows[b], aggsh.at[idxv[0].at[1]], ssem[b]).wait()

  for jj in range(4):
    issue_idx(jj, jj)
  for jj in range(GA):
    wait_idx(jj)
    issue_gather(jj, jj)

  def group(q, carry):
    for u in range(NI):
      j = q * NI + u
      b = u % NB
      issue_idx(j + 4, (u + 4) % NI)
      bn = (b + GA) % NB

      @pl.when(j + GA < nchunks)
      def _():
        @pl.when(j + GA >= NB)
        def _():
          wait_scatter(bn)
        wait_idx((u + GA) % NI)
        issue_gather((u + GA) % NI, bn)

      wait_gather(b)
      issue_scatter(u, b)
    return carry
  lax.fori_loop(0, nchunks // NI, group, 0)

  for b in range(NB):
    wait_scatter(b)
  for bi in range(4):
    wait_idx(bi)
  plsc.subcore_barrier()

  @pl.when((s == 0) & (c == 0))
  def _():
    pltpu.sync_copy(aggsh, out0)

  @pl.when((s == 0) & (c == 1))
  def _():
    pltpu.sync_copy(aggsh, out1)


def _spmm_call(g0, g1, srcp, dstp, zeros_big):
  mesh = plsc.VectorSubcoreMesh(core_axis_name="c", subcore_axis_name="s",
                                num_cores=NC, num_subcores=NS)
  f = pl.kernel(
      _spmm_body,
      out_type=(jax.ShapeDtypeStruct((NPAD, CH), jnp.float32),
                jax.ShapeDtypeStruct((NPAD, CH), jnp.float32)),
      mesh=mesh,
      scratch_types=[
          [pltpu.VMEM((2, KS), jnp.int32) for _ in range(NI)],
          [pltpu.VMEM((KS, CH), jnp.float32) for _ in range(NB)],
          pltpu.VMEM_SHARED((NPAD, CH), jnp.float32),
          [pltpu.SemaphoreType.DMA for _ in range(NI)],
          [pltpu.SemaphoreType.DMA for _ in range(NB)],
          [pltpu.SemaphoreType.DMA for _ in range(NB)],
      ],
  )
  return f(g0, g1, srcp, dstp, zeros_big)


def _fc0_body(feat, w, b, degp0, degp1, h0_o, g0_o, g1_o, dinv_o):
  deg = degp0[:, 0:1] + degp1[:, 0:1]
  dinv = lax.rsqrt(jnp.maximum(deg, 1.0))
  h0 = jnp.maximum(
      jnp.dot(feat[...], w[...], preferred_element_type=jnp.float32) + b[...],
      0.0)
  h0_o[...] = h0
  gs = h0 * dinv
  g0_o[...] = gs[:, :CH]
  g1_o[...] = gs[:, CH:]
  dinv_o[...] = dinv


def _fc0_call(features, w0t, b0, degp0, degp1):
  grid = N // BN
  return pl.pallas_call(
      _fc0_body,
      grid=(grid,),
      in_specs=[
          pl.BlockSpec((BN, F_IN), lambda i: (i, 0)),
          pl.BlockSpec((F_IN, H), lambda i: (0, 0)),
          pl.BlockSpec((1, H), lambda i: (0, 0)),
          pl.BlockSpec((BN, CH), lambda i: (i, 0)),
          pl.BlockSpec((BN, CH), lambda i: (i, 0)),
      ],
      out_specs=[
          pl.BlockSpec((BN, H), lambda i: (i, 0)),
          pl.BlockSpec((BN, CH), lambda i: (i, 0)),
          pl.BlockSpec((BN, CH), lambda i: (i, 0)),
          pl.BlockSpec((BN, 1), lambda i: (i, 0)),
      ],
      out_shape=[
          jax.ShapeDtypeStruct((N, H), jnp.float32),
          jax.ShapeDtypeStruct((N, CH), jnp.float32),
          jax.ShapeDtypeStruct((N, CH), jnp.float32),
          jax.ShapeDtypeStruct((N, 1), jnp.float32),
      ],
  )(features, w0t, b0, degp0, degp1)


def _layer_body(agg0, agg1, h0, dinv, w, o0, o1, *, beta, scale_out):
  dv = dinv[...]
  agg = jnp.concatenate([agg0[...], agg1[...]], axis=-1) * dv
  hh = (1.0 - ALPHA) * agg + ALPHA * h0[...]
  z = (1.0 - beta) * hh + beta * jnp.dot(
      hh, w[...], preferred_element_type=jnp.float32)
  h = jnp.maximum(z, 0.0)
  if scale_out:
    h = h * dv
  o0[...] = h[:, :CH]
  o1[...] = h[:, CH:]


def _layer_call(agg0, agg1, h0, dinv, w, beta, scale_out):
  grid = N // BN
  return pl.pallas_call(
      functools.partial(_layer_body, beta=beta, scale_out=scale_out),
      grid=(grid,),
      in_specs=[
          pl.BlockSpec((BN, CH), lambda i: (i, 0)),
          pl.BlockSpec((BN, CH), lambda i: (i, 0)),
          pl.BlockSpec((BN, H), lambda i: (i, 0)),
          pl.BlockSpec((BN, 1), lambda i: (i, 0)),
          pl.BlockSpec((H, H), lambda i: (0, 0)),
      ],
      out_specs=[
          pl.BlockSpec((BN, CH), lambda i: (i, 0)),
          pl.BlockSpec((BN, CH), lambda i: (i, 0)),
      ],
      out_shape=[
          jax.ShapeDtypeStruct((N, CH), jnp.float32),
          jax.ShapeDtypeStruct((N, CH), jnp.float32),
      ],
  )(agg0, agg1, h0, dinv, w)


def _fc1_body(hl, hr, w, b, out):
  hcat = jnp.concatenate([hl[...], hr[...]], axis=-1)
  out[...] = jnp.dot(hcat, w[...], preferred_element_type=jnp.float32) + b[...]


def _fc1_call(hl, hr, w1t, b1):
  grid = N // BN
  return pl.pallas_call(
      _fc1_body,
      grid=(grid,),
      in_specs=[
          pl.BlockSpec((BN, CH), lambda i: (i, 0)),
          pl.BlockSpec((BN, CH), lambda i: (i, 0)),
          pl.BlockSpec((H, 128), lambda i: (0, 0)),
          pl.BlockSpec((1, 128), lambda i: (0, 0)),
      ],
      out_specs=pl.BlockSpec((BN, 128), lambda i: (i, 0)),
      out_shape=jax.ShapeDtypeStruct((N, 128), jnp.float32),
  )(hl, hr, w1t, b1)


def kernel(features, graph, fc0_W, fc0_b, conv_W, fc1_W, fc1_b):
  src = graph[0].astype(jnp.int32)
  dst = graph[1].astype(jnp.int32)
  pad = EPAD - E
  srcp = jnp.concatenate(
      [src, jnp.zeros((pad + IDXPAD,), jnp.int32)])
  dstp = jnp.concatenate(
      [dst, jnp.full((pad,), N, jnp.int32), jnp.zeros((IDXPAD,), jnp.int32)])
  zeros_big = jnp.zeros((NPAD, CH), jnp.float32)

  degp0, degp1 = _deg_call(dstp, zeros_big)

  w0t = fc0_W.T
  b0 = fc0_b.reshape(1, H)
  h0, g0, g1, dinv = _fc0_call(features, w0t, b0, degp0, degp1)

  for i in range(L):
    beta = math.log(LAMDA / (i + 1) + 1.0)
    agg0, agg1 = _spmm_call(g0, g1, srcp, dstp, zeros_big)
    g0, g1 = _layer_call(agg0, agg1, h0, dinv, conv_W[i],
                         beta, scale_out=(i < L - 1))

  w1t = jnp.zeros((H, 128), jnp.float32).at[:, :C].set(fc1_W.T)
  b1 = jnp.zeros((1, 128), jnp.float32).at[0, :C].set(fc1_b)
  out = _fc1_call(g0, g1, w1t, b1)
  return out[:, :C]
